# baseline (speedup 1.0000x reference)
import functools

import jax
import jax.numpy as jnp
from jax import lax
from jax.experimental import pallas as pl
from jax.experimental.pallas import tpu as pltpu
from jax.experimental.pallas import tpu_sc as plsc

NUM_EMBEDDINGS = 1000000
EMBEDDING_DIM = 64
BATCH = 16384
FIELDS = 26

_INFO = plsc.get_sparse_core_info()
_NC, _NS = _INFO.num_cores, _INFO.num_subcores
_NW = _NC * _NS

_B = BATCH * FIELDS
_BPW = _B // _NW
_C = 832
_NCH = _BPW // _C
_NBUF = 2


def _body(idx_hbm, table_hbm, out_hbm, idx_v, rows_v, sems):
    wid = lax.axis_index("s") * _NC + lax.axis_index("c")
    base = wid * _BPW
    pltpu.sync_copy(idx_hbm.at[pl.ds(base, _BPW)], idx_v)

    def _gather_start(g, b):
        idx_slice = idx_v.at[pl.ds(g * _C, _C)]
        pltpu.async_copy(table_hbm.at[idx_slice], rows_v.at[b], sems.at[b])

    def _gather_wait(b):
        pltpu.make_async_copy(table_hbm.at[idx_v.at[pl.ds(0, _C)]],
                              rows_v.at[b], sems.at[b]).wait()

    def _write(g, b):
        pltpu.sync_copy(rows_v.at[b], out_hbm.at[pl.ds(base + g * _C, _C)])

    for b in range(_NBUF):
        _gather_start(b, b)

    def _step(gg):
        for b in range(_NBUF):
            g = gg + b
            _gather_wait(b)
            _write(g, b)
            _gather_start(g + _NBUF, b)

    pl.loop(0, _NCH - _NBUF, step=_NBUF)(_step)

    for b in range(_NBUF):
        _gather_wait(b)
        _write(_NCH - _NBUF + b, b)


@functools.partial(jax.jit, static_argnames=())
def kernel(inputs, weight):
    idx = inputs.reshape(_B).astype(jnp.int32)
    mesh = plsc.VectorSubcoreMesh(core_axis_name="c", subcore_axis_name="s")
    run = pl.kernel(
        _body,
        out_type=jax.ShapeDtypeStruct((_B, EMBEDDING_DIM), jnp.float32),
        mesh=mesh,
        scratch_types=[
            pltpu.VMEM((_BPW,), jnp.int32),
            pltpu.VMEM((_NBUF, _C, EMBEDDING_DIM), jnp.float32),
            pltpu.SemaphoreType.DMA((_NBUF,)),
        ],
        compiler_params=pltpu.CompilerParams(use_tc_tiling_on_sc=False),
    )
    out = run(idx, weight)
    return out.reshape(BATCH, FIELDS, EMBEDDING_DIM)


# speedup vs baseline: 1.0254x; 1.0254x over previous
import functools

import jax
import jax.numpy as jnp
from jax import lax
from jax.experimental import pallas as pl
from jax.experimental.pallas import tpu as pltpu
from jax.experimental.pallas import tpu_sc as plsc

NUM_EMBEDDINGS = 1000000
EMBEDDING_DIM = 64
PADDED_DIM = 128
BATCH = 16384
FIELDS = 26

_INFO = plsc.get_sparse_core_info()
_NC, _NS = _INFO.num_cores, _INFO.num_subcores
_NW = _NC * _NS

_B = BATCH * FIELDS
_BPW = _B // _NW
_C = 416
_NCH = _BPW // _C
_NBUF = 2


def _body(idx_hbm, table_hbm, out_hbm, idx_v, rows_v, sems):
    wid = lax.axis_index("s") * _NC + lax.axis_index("c")
    base = wid * _BPW
    pltpu.sync_copy(idx_hbm.at[pl.ds(base, _BPW)], idx_v)

    def _gather_start(g, b):
        idx_slice = idx_v.at[pl.ds(g * _C, _C)]
        pltpu.async_copy(table_hbm.at[idx_slice], rows_v.at[b], sems.at[b])

    def _gather_wait(b):
        pltpu.make_async_copy(table_hbm.at[idx_v.at[pl.ds(0, _C)]],
                              rows_v.at[b], sems.at[b]).wait()

    def _write(g, b):
        pltpu.sync_copy(rows_v.at[b, :, pl.ds(0, EMBEDDING_DIM)],
                        out_hbm.at[pl.ds(base + g * _C, _C)])

    for b in range(_NBUF):
        _gather_start(b, b)

    def _step(gg):
        for b in range(_NBUF):
            g = gg + b
            _gather_wait(b)
            _write(g, b)
            _gather_start(g + _NBUF, b)

    pl.loop(0, _NCH - _NBUF, step=_NBUF)(_step)

    for b in range(_NBUF):
        _gather_wait(b)
        _write(_NCH - _NBUF + b, b)


@functools.partial(jax.jit, static_argnames=())
def kernel(inputs, weight):
    idx = inputs.reshape(_B).astype(jnp.int32)
    w128 = jnp.pad(weight, ((0, 0), (0, PADDED_DIM - EMBEDDING_DIM)))
    mesh = plsc.VectorSubcoreMesh(core_axis_name="c", subcore_axis_name="s")
    run = pl.kernel(
        _body,
        out_type=jax.ShapeDtypeStruct((_B, EMBEDDING_DIM), jnp.float32),
        mesh=mesh,
        scratch_types=[
            pltpu.VMEM((_BPW,), jnp.int32),
            pltpu.VMEM((_NBUF, _C, PADDED_DIM), jnp.float32),
            pltpu.SemaphoreType.DMA((_NBUF,)),
        ],
        compiler_params=pltpu.CompilerParams(use_tc_tiling_on_sc=False),
    )
    out = run(idx, w128)
    return out.reshape(BATCH, FIELDS, EMBEDDING_DIM)


# speedup vs baseline: 1.0714x; 1.0449x over previous
import functools

import jax
import jax.numpy as jnp
from jax import lax
from jax.experimental import pallas as pl
from jax.experimental.pallas import tpu as pltpu
from jax.experimental.pallas import tpu_sc as plsc

NUM_EMBEDDINGS = 1000000
EMBEDDING_DIM = 64
PADDED_DIM = 128
BATCH = 16384
FIELDS = 26

_INFO = plsc.get_sparse_core_info()
_NC, _NS = _INFO.num_cores, _INFO.num_subcores
_NW = _NC * _NS

_B = BATCH * FIELDS
_BPW = _B // _NW
_C = 832
_NCH = _BPW // _C
_NBUF = 2


def _gather_body(idx_hbm, table_hbm, out_hbm, idx_v, rows_v, sems):
    wid = lax.axis_index("s") * _NC + lax.axis_index("c")
    base = wid * _BPW
    pltpu.sync_copy(idx_hbm.at[pl.ds(base, _BPW)], idx_v)

    def _gather_start(g, b):
        idx_slice = idx_v.at[pl.ds(g * _C, _C)]
        pltpu.async_copy(table_hbm.at[idx_slice], rows_v.at[b], sems.at[b])

    def _gather_wait(b):
        pltpu.make_async_copy(table_hbm.at[idx_v.at[pl.ds(0, _C)]],
                              rows_v.at[b], sems.at[b]).wait()

    def _write(g, b):
        pltpu.sync_copy(rows_v.at[b], out_hbm.at[pl.ds(base + g * _C, _C)])

    for b in range(_NBUF):
        _gather_start(b, b)

    def _step(gg):
        for b in range(_NBUF):
            g = gg + b
            _gather_wait(b)
            _write(g, b)
            _gather_start(g + _NBUF, b)

    pl.loop(0, _NCH - _NBUF, step=_NBUF)(_step)

    for b in range(_NBUF):
        _gather_wait(b)
        _write(_NCH - _NBUF + b, b)


@functools.partial(jax.jit, static_argnames=())
def kernel(inputs, weight):
    idx = inputs.reshape(_B).astype(jnp.int32) * 2
    table = jnp.pad(weight, ((0, 0), (0, PADDED_DIM - EMBEDDING_DIM)))
    table2 = table.reshape(2 * NUM_EMBEDDINGS, EMBEDDING_DIM)
    mesh = plsc.VectorSubcoreMesh(core_axis_name="c", subcore_axis_name="s")
    gather_run = pl.kernel(
        _gather_body,
        out_type=jax.ShapeDtypeStruct((_B, EMBEDDING_DIM), jnp.float32),
        mesh=mesh,
        scratch_types=[
            pltpu.VMEM((_BPW,), jnp.int32),
            pltpu.VMEM((_NBUF, _C, EMBEDDING_DIM), jnp.float32),
            pltpu.SemaphoreType.DMA((_NBUF,)),
        ],
        compiler_params=pltpu.CompilerParams(use_tc_tiling_on_sc=False),
    )
    out = gather_run(idx, table2)
    return out.reshape(BATCH, FIELDS, EMBEDDING_DIM)
